# baseline (speedup 1.0000x reference)
import jax
import jax.numpy as jnp
from jax.experimental import pallas as pl

G = 128
N = 64
F_IN = 128
EMB = 128
GB = 128

_NEG_INF = float("-inf")
_BATCH_DN = (((1,), (1,)), ((0,), (0,)))


def _softmax(x):
    m = jnp.max(x, axis=-1, keepdims=True)
    e = jnp.exp(x - m)
    return e / jnp.sum(e, axis=-1, keepdims=True)


def _sample(probs, gumbel, width):
    score = jnp.log(probs + 1e-20) + gumbel
    m = jnp.max(score, axis=-1, keepdims=True)
    idx = jax.lax.broadcasted_iota(jnp.int32, score.shape, score.ndim - 1)
    cand = jnp.where(score == m, idx, width)
    return jnp.min(cand, axis=-1, keepdims=True)


def _actor_body(states_ref, g1_ref, g2_ref, g3_ref,
                W1a_ref, b1a_ref, W1b_ref, b1b_ref,
                W2a_ref, b2a_ref, W2b_ref, b2b_ref,
                W3a_ref, b3a_ref, W3b_ref, b3b_ref,
                WA1_ref, bA1_ref, WA2_ref, bA2_ref,
                WB1_ref, bB1_ref, WB2_ref, bB2_ref,
                WC1_ref, bC1_ref, WC2_ref, bC2_ref,
                pA_ref, pB_ref, pC_ref, act_ref):
    f32 = jnp.float32
    s = states_ref[...]
    A = (s > 0).astype(f32)
    amask_f = (jnp.sum(s, axis=1) > 0).astype(f32)

    xin = (jax.lax.dot_general(A, jnp.ones((GB, N, F_IN), f32), _BATCH_DN)
           + 1.0).reshape(GB * N, F_IN)
    h = jax.nn.relu(jnp.dot(xin, W1a_ref[...]) + b1a_ref[...])
    x = jax.nn.relu(jnp.dot(h, W1b_ref[...]) + b1b_ref[...])

    for Wa_ref, ba_ref, Wb_ref, bb_ref in (
            (W2a_ref, b2a_ref, W2b_ref, b2b_ref),
            (W3a_ref, b3a_ref, W3b_ref, b3b_ref)):
        x3 = x.reshape(GB, N, EMB)
        agg = jax.lax.dot_general(A, x3, _BATCH_DN)
        hin = (x3 + agg).reshape(GB * N, EMB)
        h = jax.nn.relu(jnp.dot(hin, Wa_ref[...]) + ba_ref[...])
        x = jax.nn.relu(jnp.dot(h, Wb_ref[...]) + bb_ref[...])

    x3 = x.reshape(GB, N, EMB)

    ri = jax.lax.broadcasted_iota(jnp.int32, (N, N), 0)
    ci = jax.lax.broadcasted_iota(jnp.int32, (N, N), 1)
    U = (ri <= ci).astype(f32)
    cum = jnp.dot(amask_f, U)
    count = jnp.sum(amask_f, axis=1, keepdims=True)
    ns_f = amask_f * (cum < count).astype(f32)
    rank_ns = jnp.dot(ns_f, U) - 1.0
    rank_a = cum - 1.0

    hA = jax.nn.relu(jnp.dot(x, WA1_ref[...]) + bA1_ref[...])
    wa2 = jnp.reshape(WA2_ref[...], (1, 1, 32))
    la = (jnp.sum(hA.reshape(GB, N, 32) * wa2, axis=2)
          + bA2_ref[...])
    kA = jax.lax.broadcasted_iota(jnp.int32, (1, 1, N - 1), 2).astype(f32)
    PA = (rank_ns[:, :, None] == kA).astype(f32) * ns_f[:, :, None]
    lAc = jax.lax.dot_general(la, PA, _BATCH_DN)
    ns_count = jnp.sum(ns_f, axis=1, keepdims=True)
    vA = jax.lax.broadcasted_iota(jnp.int32, (GB, N - 1), 1).astype(f32) < ns_count
    pA = _softmax(jnp.where(vA, lAc, _NEG_INF))
    first_sel = _sample(pA, g1_ref[...], N - 1)

    onehot = (jax.lax.broadcasted_iota(jnp.int32, (GB, N), 1)
              == first_sel).astype(f32)
    fe = jax.lax.dot_general(onehot, x3, _BATCH_DN)
    xb = jnp.dot(x, WB1_ref[0:EMB, :]).reshape(GB, N, 32)
    feb = jnp.dot(fe, WB1_ref[EMB:2 * EMB, :])[:, None, :]
    hB = jax.nn.relu(xb + feb + bB1_ref[...])
    wb2 = jnp.reshape(WB2_ref[...], (1, 1, 32))
    lb = jnp.sum(hB * wb2, axis=2) + bB2_ref[...]
    kB = jax.lax.broadcasted_iota(jnp.int32, (1, 1, N), 2).astype(f32)
    PB = (rank_a[:, :, None] == kB).astype(f32) * amask_f[:, :, None]
    lBc = jax.lax.dot_general(lb, PB, _BATCH_DN)
    vB = jax.lax.broadcasted_iota(jnp.int32, (GB, N), 1).astype(f32) < count
    pB = _softmax(jnp.where(vB, lBc, _NEG_INF))
    second_sel = _sample(pB, g2_ref[...], N)

    sums = jax.lax.dot_general(amask_f, x3, _BATCH_DN)
    gemb = sums / jnp.maximum(count, 1.0)
    hC = jax.nn.relu(jnp.dot(gemb, WC1_ref[...]) + bC1_ref[...])
    lc = jnp.dot(hC, WC2_ref[...]) + bC2_ref[...]
    pC = _softmax(lc)
    is_end = _sample(pC, g3_ref[...], 2)

    pA_ref[...] = pA
    pB_ref[...] = pB
    pC_ref[...] = pC
    act_ref[...] = jnp.concatenate([first_sel, second_sel, is_end], axis=1)


def kernel(states, W1a, b1a, W1b, b1b, W2a, b2a, W2b, b2b,
           W3a, b3a, W3b, b3b, WA1, bA1, WA2, bA2,
           WB1, bB1, WB2, bB2, WC1, bC1, WC2, bC2):
    f32 = jnp.float32
    g1 = jax.random.gumbel(jax.random.key(1), (G, N - 1), f32)
    g2 = jax.random.gumbel(jax.random.key(2), (G, N), f32)
    g3 = jax.random.gumbel(jax.random.key(3), (G, 2), f32)

    def blk(shape):
        return pl.BlockSpec(shape, lambda i: (i,) + (0,) * (len(shape) - 1))

    def full(shape):
        return pl.BlockSpec(shape, lambda i: (0,) * len(shape))

    in_specs = [
        blk((GB, N, N)),
        blk((GB, N - 1)), blk((GB, N)), blk((GB, 2)),
        full((F_IN, EMB)), full((EMB,)), full((EMB, EMB)), full((EMB,)),
        full((EMB, EMB)), full((EMB,)), full((EMB, EMB)), full((EMB,)),
        full((EMB, EMB)), full((EMB,)), full((EMB, EMB)), full((EMB,)),
        full((EMB, 32)), full((32,)), full((32, 1)), full((1,)),
        full((2 * EMB, 32)), full((32,)), full((32, 1)), full((1,)),
        full((EMB, 32)), full((32,)), full((32, 2)), full((2,)),
    ]
    out_specs = (blk((GB, N - 1)), blk((GB, N)), blk((GB, 2)), blk((GB, 3)))
    out_shape = (
        jax.ShapeDtypeStruct((G, N - 1), f32),
        jax.ShapeDtypeStruct((G, N), f32),
        jax.ShapeDtypeStruct((G, 2), f32),
        jax.ShapeDtypeStruct((G, 3), jnp.int32),
    )

    pA, pB, pC, act = pl.pallas_call(
        _actor_body,
        grid=(G // GB,),
        in_specs=in_specs,
        out_specs=out_specs,
        out_shape=out_shape,
    )(states, g1, g2, g3,
      W1a, b1a, W1b, b1b, W2a, b2a, W2b, b2b, W3a, b3a, W3b, b3b,
      WA1, bA1, WA2, bA2, WB1, bB1, WB2, bB2, WC1, bC1, WC2, bC2)
    return pA, pB, pC, act


# speedup vs baseline: 1.0682x; 1.0682x over previous
import jax
import jax.numpy as jnp
from jax.experimental import pallas as pl

G = 128
N = 64
F_IN = 128
EMB = 128
GB = 128

_NEG_INF = float("-inf")
_BATCH_DN = (((1,), (1,)), ((0,), (0,)))


def _softmax(x):
    m = jnp.max(x, axis=-1, keepdims=True)
    e = jnp.exp(x - m)
    return e / jnp.sum(e, axis=-1, keepdims=True)


def _sample(probs, gumbel, width):
    score = jnp.log(probs + 1e-20) + gumbel
    m = jnp.max(score, axis=-1, keepdims=True)
    idx = jax.lax.broadcasted_iota(jnp.int32, score.shape, score.ndim - 1)
    cand = jnp.where(score == m, idx, width)
    return jnp.min(cand, axis=-1, keepdims=True)


def _actor_body(states_ref, g1_ref, g2_ref, g3_ref,
                W1a_ref, b1a_ref, W1b_ref, b1b_ref,
                W2a_ref, b2a_ref, W2b_ref, b2b_ref,
                W3a_ref, b3a_ref, W3b_ref, b3b_ref,
                WA1_ref, bA1_ref, WA2_ref, bA2_ref,
                WB1_ref, bB1_ref, WB2_ref, bB2_ref,
                WC1_ref, bC1_ref, WC2_ref, bC2_ref,
                pA_ref, pB_ref, pC_ref, act_ref):
    f32 = jnp.float32
    s = states_ref[...]
    A = (s > 0).astype(f32)
    amask_f = (jnp.sum(s, axis=1) > 0).astype(f32)

    xin = (jax.lax.dot_general(A, jnp.ones((GB, N, F_IN), f32), _BATCH_DN)
           + 1.0).reshape(GB * N, F_IN)
    h = jax.nn.relu(jnp.dot(xin, W1a_ref[...]) + b1a_ref[...])
    x = jax.nn.relu(jnp.dot(h, W1b_ref[...]) + b1b_ref[...])

    for Wa_ref, ba_ref, Wb_ref, bb_ref in (
            (W2a_ref, b2a_ref, W2b_ref, b2b_ref),
            (W3a_ref, b3a_ref, W3b_ref, b3b_ref)):
        x3 = x.reshape(GB, N, EMB)
        agg = jax.lax.dot_general(A, x3, _BATCH_DN)
        hin = (x3 + agg).reshape(GB * N, EMB)
        h = jax.nn.relu(jnp.dot(hin, Wa_ref[...]) + ba_ref[...])
        x = jax.nn.relu(jnp.dot(h, Wb_ref[...]) + bb_ref[...])

    x3 = x.reshape(GB, N, EMB)

    ri = jax.lax.broadcasted_iota(jnp.int32, (N, N), 0)
    ci = jax.lax.broadcasted_iota(jnp.int32, (N, N), 1)
    U = (ri <= ci).astype(f32)
    cum = jnp.dot(amask_f, U)
    count = jnp.sum(amask_f, axis=1, keepdims=True)
    ns_f = amask_f * (cum < count).astype(f32)
    rank_ns = jnp.dot(ns_f, U) - 1.0
    rank_a = cum - 1.0

    hA = jax.nn.relu(jnp.dot(x, WA1_ref[...]) + bA1_ref[...])
    la = (jnp.sum(hA.reshape(GB, N, 32) * WA2_ref[...][None], axis=2)
          + bA2_ref[...])
    kA = jax.lax.broadcasted_iota(jnp.int32, (1, 1, N - 1), 2).astype(f32)
    PA = (rank_ns[:, :, None] == kA).astype(f32) * ns_f[:, :, None]
    lAc = jax.lax.dot_general(la, PA, _BATCH_DN)
    ns_count = jnp.sum(ns_f, axis=1, keepdims=True)
    vA = jax.lax.broadcasted_iota(jnp.int32, (GB, N - 1), 1).astype(f32) < ns_count
    pA = _softmax(jnp.where(vA, lAc, _NEG_INF))
    first_sel = _sample(pA, g1_ref[...], N - 1)

    onehot = (jax.lax.broadcasted_iota(jnp.int32, (GB, N), 1)
              == first_sel).astype(f32)
    fe = jax.lax.dot_general(onehot, x3, _BATCH_DN)
    xb = jnp.dot(x, WB1_ref[0:EMB, :]).reshape(GB, N, 32)
    feb = jnp.dot(fe, WB1_ref[EMB:2 * EMB, :])[:, None, :]
    hB = jax.nn.relu(xb + feb + bB1_ref[...])
    lb = jnp.sum(hB * WB2_ref[...][None], axis=2) + bB2_ref[...]
    kB = jax.lax.broadcasted_iota(jnp.int32, (1, 1, N), 2).astype(f32)
    PB = (rank_a[:, :, None] == kB).astype(f32) * amask_f[:, :, None]
    lBc = jax.lax.dot_general(lb, PB, _BATCH_DN)
    vB = jax.lax.broadcasted_iota(jnp.int32, (GB, N), 1).astype(f32) < count
    pB = _softmax(jnp.where(vB, lBc, _NEG_INF))
    second_sel = _sample(pB, g2_ref[...], N)

    sums = jax.lax.dot_general(amask_f, x3, _BATCH_DN)
    gemb = sums / jnp.maximum(count, 1.0)
    hC = jax.nn.relu(jnp.dot(gemb, WC1_ref[...]) + bC1_ref[...])
    lc = jnp.dot(hC, WC2_ref[...]) + bC2_ref[...]
    pC = _softmax(lc)
    is_end = _sample(pC, g3_ref[...], 2)

    pA_ref[...] = pA
    pB_ref[...] = pB
    pC_ref[...] = pC
    act_ref[...] = jnp.concatenate([first_sel, second_sel, is_end], axis=1)


def kernel(states, W1a, b1a, W1b, b1b, W2a, b2a, W2b, b2b,
           W3a, b3a, W3b, b3b, WA1, bA1, WA2, bA2,
           WB1, bB1, WB2, bB2, WC1, bC1, WC2, bC2):
    f32 = jnp.float32
    g1 = jax.random.gumbel(jax.random.key(1), (G, N - 1), f32)
    g2 = jax.random.gumbel(jax.random.key(2), (G, N), f32)
    g3 = jax.random.gumbel(jax.random.key(3), (G, 2), f32)

    def blk(shape):
        return pl.BlockSpec(shape, lambda i: (i,) + (0,) * (len(shape) - 1))

    def full(shape):
        return pl.BlockSpec(shape, lambda i: (0,) * len(shape))

    in_specs = [
        blk((GB, N, N)),
        blk((GB, N - 1)), blk((GB, N)), blk((GB, 2)),
        full((F_IN, EMB)), full((1, EMB)), full((EMB, EMB)), full((1, EMB)),
        full((EMB, EMB)), full((1, EMB)), full((EMB, EMB)), full((1, EMB)),
        full((EMB, EMB)), full((1, EMB)), full((EMB, EMB)), full((1, EMB)),
        full((EMB, 32)), full((1, 32)), full((1, 32)), full((1, 1)),
        full((2 * EMB, 32)), full((1, 32)), full((1, 32)), full((1, 1)),
        full((EMB, 32)), full((1, 32)), full((32, 2)), full((1, 2)),
    ]
    out_specs = (blk((GB, N - 1)), blk((GB, N)), blk((GB, 2)), blk((GB, 3)))
    out_shape = (
        jax.ShapeDtypeStruct((G, N - 1), f32),
        jax.ShapeDtypeStruct((G, N), f32),
        jax.ShapeDtypeStruct((G, 2), f32),
        jax.ShapeDtypeStruct((G, 3), jnp.int32),
    )

    pA, pB, pC, act = pl.pallas_call(
        _actor_body,
        grid=(G // GB,),
        in_specs=in_specs,
        out_specs=out_specs,
        out_shape=out_shape,
    )(states, g1, g2, g3,
      W1a, b1a.reshape(1, EMB), W1b, b1b.reshape(1, EMB),
      W2a, b2a.reshape(1, EMB), W2b, b2b.reshape(1, EMB),
      W3a, b3a.reshape(1, EMB), W3b, b3b.reshape(1, EMB),
      WA1, bA1.reshape(1, 32), WA2.reshape(1, 32), bA2.reshape(1, 1),
      WB1, bB1.reshape(1, 32), WB2.reshape(1, 32), bB2.reshape(1, 1),
      WC1, bC1.reshape(1, 32), WC2, bC2.reshape(1, 2))
    return pA, pB, pC, act
